# baseline (speedup 1.0000x reference)
import functools

import jax
import jax.numpy as jnp
import numpy as np
from jax import lax
from jax.experimental import pallas as pl
from jax.experimental.pallas import tpu as pltpu
from jax.experimental.pallas import tpu_sc as plsc

N_POINTS = 200
SIDE = 33
NV = SIDE ** 3
NV_PAD = 35952
NC, NS = 2, 16
NW = NC * NS
LANES = 16
B = 8192
RPT = B // NW
NG = RPT // LANES
DT = np.float32(1.0 / (N_POINTS - 1))
POS_HI = np.float32(np.float32(1.0 - 1e-6) * 32.0)

_CORNER_OFF = (0, 1, 33, 34, 1089, 1090, 1122, 1123)


def _interp_matrix(R):
    i = jnp.arange(SIDE, dtype=jnp.float32)
    pos = i * np.float32(R / 32.0)
    pi = jnp.clip(jnp.floor(pos).astype(jnp.int32), 0, R - 1)
    frac = pos - pi.astype(jnp.float32)
    lo = jax.nn.one_hot(pi, R + 1, dtype=jnp.float32) * (1.0 - frac)[:, None]
    hi = jax.nn.one_hot(pi + 1, R + 1, dtype=jnp.float32) * frac[:, None]
    return lo + hi


def _upsample_fm(grid, R):
    gT = grid.T.reshape(4, R + 1, R + 1, R + 1)
    W = _interp_matrix(R)
    gT = jnp.einsum("ai,fijk->fajk", W, gT)
    gT = jnp.einsum("bj,fajk->fabk", W, gT)
    gT = jnp.einsum("ck,fabk->fabc", W, gT)
    return gT.reshape(4, NV)


def _build_fields(grid0, grid1, grid2, W_label, b_label, W_rgb, b_rgb):
    U0 = _upsample_fm(grid0, 8)
    U1 = _upsample_fm(grid1, 16)
    U2 = (grid2[:, 0], grid2[:, 1], grid2[:, 2], grid2[:, 3])
    U = (U0, U1, U2)
    lab = (b_label[0] + W_label[0, 0] * U0[0] + W_label[1, 0] * U1[0]
           + W_label[2, 0] * U2[0])
    chans = [lab]
    for c in range(3):
        chans.append(b_rgb[c] + sum(W_rgb[3 * l + f - 1, c] * U[l][f]
                                    for l in range(3) for f in (1, 2, 3)))
    z = jnp.zeros((NV_PAD - NV,), jnp.float32)
    return jnp.concatenate([a for ch in chans for a in (ch, z)])


def _sigmoid(x):
    return 1.0 / (1.0 + jnp.exp(-x))


def _sc_body(rdat_hbm, flds_hbm, hits_hbm, rgb_hbm,
             rdat_v, fld_v, idx0_v, fsel_v, hits_st, rgb_st):
    wid = lax.axis_index("s") * NC + lax.axis_index("c")
    base = wid * RPT
    for i in range(6):
        pltpu.sync_copy(rdat_hbm.at[pl.ds(i * B + base, RPT)],
                        rdat_v.at[pl.ds(i * RPT, RPT)])
    pltpu.sync_copy(flds_hbm.at[pl.ds(0, NV_PAD)], fld_v.at[pl.ds(0, NV_PAD)])
    iota = lax.iota(jnp.int32, LANES)

    for g in range(NG):
        sl = pl.ds(g * LANES, LANES)
        p1 = tuple(rdat_v[pl.ds(i * RPT + g * LANES, LANES)] for i in range(3))
        dd = tuple(rdat_v[pl.ds(i * RPT + g * LANES, LANES)] for i in range(3, 6))

        def cell(tj, p1=p1, dd=dd):
            pifs = []
            for p1c, dc in zip(p1, dd):
                pos = jnp.minimum(
                    jnp.maximum((p1c + dc * tj + 1.0) * 16.0, 0.0), POS_HI)
                piv = pos.astype(jnp.int32)
                pifs.append((piv, pos - piv.astype(jnp.float32)))
            (pix, fx), (piy, fy), (piz, fz) = pifs
            return (pix * 33 + piy) * 33 + piz, fx, fy, fz

        def trilerp(vals, fx, fy, fz):
            a00 = vals[0] + (vals[1] - vals[0]) * fz
            a01 = vals[2] + (vals[3] - vals[2]) * fz
            a10 = vals[4] + (vals[5] - vals[4]) * fz
            a11 = vals[6] + (vals[7] - vals[6]) * fz
            b0 = a00 + (a01 - a00) * fy
            b1 = a10 + (a11 - a10) * fy
            return b0 + (b1 - b0) * fx

        def body(j, carry, cell=cell, trilerp=trilerp):
            vmax, vmin = carry
            tj = jnp.full((LANES,), j.astype(jnp.float32) * DT)
            idx0, fx, fy, fz = cell(tj)
            vals = [plsc.load_gather(fld_v, [idx0 + off]) for off in _CORNER_OFF]
            lg = trilerp(vals, fx, fy, fz)
            vmax = jnp.maximum(vmax, lg)
            cand = jnp.where(lg > 0.0, jnp.full((LANES,), j, jnp.int32),
                             jnp.full((LANES,), N_POINTS, jnp.int32))
            return vmax, jnp.minimum(vmin, cand)

        init = (jnp.full((LANES,), -jnp.inf, jnp.float32),
                jnp.full((LANES,), N_POINTS, jnp.int32))
        vmax, vmin = lax.fori_loop(0, N_POINTS, body, init)

        hits_st[sl] = _sigmoid(vmax)
        idx_sel = jnp.where(vmin == N_POINTS, jnp.zeros((LANES,), jnp.int32), vmin)
        idx0, fx, fy, fz = cell(idx_sel.astype(jnp.float32) * DT)
        idx0_v[sl] = idx0
        fsel_v[pl.ds(0 * RPT + g * LANES, LANES)] = fx
        fsel_v[pl.ds(1 * RPT + g * LANES, LANES)] = fy
        fsel_v[pl.ds(2 * RPT + g * LANES, LANES)] = fz

    pltpu.sync_copy(flds_hbm.at[pl.ds(NV_PAD, 3 * NV_PAD)], fld_v)

    for g in range(NG):
        sl = pl.ds(g * LANES, LANES)
        idx0 = idx0_v[sl]
        fx = fsel_v[pl.ds(0 * RPT + g * LANES, LANES)]
        fy = fsel_v[pl.ds(1 * RPT + g * LANES, LANES)]
        fz = fsel_v[pl.ds(2 * RPT + g * LANES, LANES)]
        r3 = (iota + g * LANES) * 3
        for c in range(3):
            vals = [plsc.load_gather(fld_v, [idx0 + (c * NV_PAD + off)])
                    for off in _CORNER_OFF]
            a00 = vals[0] + (vals[1] - vals[0]) * fz
            a01 = vals[2] + (vals[3] - vals[2]) * fz
            a10 = vals[4] + (vals[5] - vals[4]) * fz
            a11 = vals[6] + (vals[7] - vals[6]) * fz
            b0 = a00 + (a01 - a00) * fy
            b1 = a10 + (a11 - a10) * fy
            plsc.store_scatter(rgb_st, [r3 + c], _sigmoid(b0 + (b1 - b0) * fx))

    pltpu.sync_copy(hits_st, hits_hbm.at[pl.ds(base, RPT)])
    pltpu.sync_copy(rgb_st, rgb_hbm.at[pl.ds(base * 3, 3 * RPT)])


@functools.cache
def _get_sc_kernel():
    return functools.partial(
        pl.kernel,
        out_type=(jax.ShapeDtypeStruct((B,), jnp.float32),
                  jax.ShapeDtypeStruct((3 * B,), jnp.float32)),
        mesh=plsc.VectorSubcoreMesh(core_axis_name="c", subcore_axis_name="s",
                                    num_cores=NC, num_subcores=NS),
        compiler_params=pltpu.CompilerParams(needs_layout_passes=False,
                                             use_tc_tiling_on_sc=False),
        scratch_types=[
            pltpu.VMEM((6 * RPT,), jnp.float32),
            pltpu.VMEM((3 * NV_PAD,), jnp.float32),
            pltpu.VMEM((RPT,), jnp.int32),
            pltpu.VMEM((3 * RPT,), jnp.float32),
            pltpu.VMEM((RPT,), jnp.float32),
            pltpu.VMEM((3 * RPT,), jnp.float32),
        ],
    )(_sc_body)


def kernel(x, grid0, grid1, grid2, W_label, b_label, W_rgb, b_rgb):
    st1, ct1 = jnp.sin(x[:, 0]), jnp.cos(x[:, 0])
    st2, ct2 = jnp.sin(x[:, 2]), jnp.cos(x[:, 2])
    p1x, p1y, p1z = st1 * jnp.cos(x[:, 1]), st1 * jnp.sin(x[:, 1]), ct1
    p2x, p2y, p2z = st2 * jnp.cos(x[:, 3]), st2 * jnp.sin(x[:, 3]), ct2
    rdat = jnp.concatenate(
        [p1x, p1y, p1z, p2x - p1x, p2y - p1y, p2z - p1z])
    flds = _build_fields(grid0, grid1, grid2, W_label, b_label, W_rgb, b_rgb)
    hits_flat, rgb_flat = _get_sc_kernel()(rdat, flds)
    return hits_flat.reshape(B, 1), rgb_flat.reshape(B, 3)


# speedup vs baseline: 1.0876x; 1.0876x over previous
import functools

import jax
import jax.numpy as jnp
import numpy as np
from jax import lax
from jax.experimental import pallas as pl
from jax.experimental.pallas import tpu as pltpu
from jax.experimental.pallas import tpu_sc as plsc

N_POINTS = 200
SIDE = 33
NV = SIDE ** 3
NV_PAD = 35952
NC, NS = 2, 16
NW = NC * NS
LANES = 16
B = 8192
RPT = B // NW
NG = RPT // LANES
DT = np.float32(1.0 / (N_POINTS - 1))
POS_HI = np.float32(np.float32(1.0 - 1e-6) * 32.0)

_CORNER_OFF = (0, 1, 33, 34, 1089, 1090, 1122, 1123)


def _interp_matrix(R):
    i = jnp.arange(SIDE, dtype=jnp.float32)
    pos = i * np.float32(R / 32.0)
    pi = jnp.clip(jnp.floor(pos).astype(jnp.int32), 0, R - 1)
    frac = pos - pi.astype(jnp.float32)
    lo = jax.nn.one_hot(pi, R + 1, dtype=jnp.float32) * (1.0 - frac)[:, None]
    hi = jax.nn.one_hot(pi + 1, R + 1, dtype=jnp.float32) * frac[:, None]
    return lo + hi


def _upsample_fm(grid, R):
    gT = grid.T.reshape(4, R + 1, R + 1, R + 1)
    W = _interp_matrix(R)
    gT = jnp.einsum("ai,fijk->fajk", W, gT)
    gT = jnp.einsum("bj,fajk->fabk", W, gT)
    gT = jnp.einsum("ck,fabk->fabc", W, gT)
    return gT.reshape(4, NV)


def _build_fields(grid0, grid1, grid2, W_label, b_label, W_rgb, b_rgb):
    U0 = lax.optimization_barrier(_upsample_fm(grid0, 8))
    U1 = lax.optimization_barrier(_upsample_fm(grid1, 16))
    U2 = lax.optimization_barrier(grid2.T)
    U = (U0, U1, U2)
    lab = (b_label[0] + W_label[0, 0] * U0[0] + W_label[1, 0] * U1[0]
           + W_label[2, 0] * U2[0])
    chans = [lab]
    for c in range(3):
        chans.append(b_rgb[c] + sum(W_rgb[3 * l + f - 1, c] * U[l][f]
                                    for l in range(3) for f in (1, 2, 3)))
    z = jnp.zeros((NV_PAD - NV,), jnp.float32)
    return jnp.concatenate([a for ch in chans for a in (ch, z)])


def _sigmoid(x):
    return 1.0 / (1.0 + jnp.exp(-x))


def _sc_body(rdat_hbm, flds_hbm, hits_hbm, rgb_hbm,
             rdat_v, fld_v, idx0_v, fsel_v, hits_st, rgb_st):
    wid = lax.axis_index("s") * NC + lax.axis_index("c")
    base = wid * RPT
    for i in range(6):
        pltpu.sync_copy(rdat_hbm.at[pl.ds(i * B + base, RPT)],
                        rdat_v.at[pl.ds(i * RPT, RPT)])
    pltpu.sync_copy(flds_hbm.at[pl.ds(0, NV_PAD)], fld_v.at[pl.ds(0, NV_PAD)])
    iota = lax.iota(jnp.int32, LANES)

    for g in range(NG):
        sl = pl.ds(g * LANES, LANES)
        p1 = tuple(rdat_v[pl.ds(i * RPT + g * LANES, LANES)] for i in range(3))
        dd = tuple(rdat_v[pl.ds(i * RPT + g * LANES, LANES)] for i in range(3, 6))

        def cell(tj, p1=p1, dd=dd):
            pifs = []
            for p1c, dc in zip(p1, dd):
                pos = jnp.minimum(
                    jnp.maximum((p1c + dc * tj + 1.0) * 16.0, 0.0), POS_HI)
                piv = pos.astype(jnp.int32)
                pifs.append((piv, pos - piv.astype(jnp.float32)))
            (pix, fx), (piy, fy), (piz, fz) = pifs
            return (pix * 33 + piy) * 33 + piz, fx, fy, fz

        def trilerp(vals, fx, fy, fz):
            a00 = vals[0] + (vals[1] - vals[0]) * fz
            a01 = vals[2] + (vals[3] - vals[2]) * fz
            a10 = vals[4] + (vals[5] - vals[4]) * fz
            a11 = vals[6] + (vals[7] - vals[6]) * fz
            b0 = a00 + (a01 - a00) * fy
            b1 = a10 + (a11 - a10) * fy
            return b0 + (b1 - b0) * fx

        def body(j, carry, cell=cell, trilerp=trilerp):
            vmax, vmin = carry
            tj = jnp.full((LANES,), j.astype(jnp.float32) * DT)
            idx0, fx, fy, fz = cell(tj)
            vals = [plsc.load_gather(fld_v, [idx0 + off]) for off in _CORNER_OFF]
            lg = trilerp(vals, fx, fy, fz)
            vmax = jnp.maximum(vmax, lg)
            cand = jnp.where(lg > 0.0, jnp.full((LANES,), j, jnp.int32),
                             jnp.full((LANES,), N_POINTS, jnp.int32))
            return vmax, jnp.minimum(vmin, cand)

        init = (jnp.full((LANES,), -jnp.inf, jnp.float32),
                jnp.full((LANES,), N_POINTS, jnp.int32))
        vmax, vmin = lax.fori_loop(0, N_POINTS, body, init)

        hits_st[sl] = _sigmoid(vmax)
        idx_sel = jnp.where(vmin == N_POINTS, jnp.zeros((LANES,), jnp.int32), vmin)
        idx0, fx, fy, fz = cell(idx_sel.astype(jnp.float32) * DT)
        idx0_v[sl] = idx0
        fsel_v[pl.ds(0 * RPT + g * LANES, LANES)] = fx
        fsel_v[pl.ds(1 * RPT + g * LANES, LANES)] = fy
        fsel_v[pl.ds(2 * RPT + g * LANES, LANES)] = fz

    pltpu.sync_copy(flds_hbm.at[pl.ds(NV_PAD, 3 * NV_PAD)], fld_v)

    for g in range(NG):
        sl = pl.ds(g * LANES, LANES)
        idx0 = idx0_v[sl]
        fx = fsel_v[pl.ds(0 * RPT + g * LANES, LANES)]
        fy = fsel_v[pl.ds(1 * RPT + g * LANES, LANES)]
        fz = fsel_v[pl.ds(2 * RPT + g * LANES, LANES)]
        r3 = (iota + g * LANES) * 3
        for c in range(3):
            vals = [plsc.load_gather(fld_v, [idx0 + (c * NV_PAD + off)])
                    for off in _CORNER_OFF]
            a00 = vals[0] + (vals[1] - vals[0]) * fz
            a01 = vals[2] + (vals[3] - vals[2]) * fz
            a10 = vals[4] + (vals[5] - vals[4]) * fz
            a11 = vals[6] + (vals[7] - vals[6]) * fz
            b0 = a00 + (a01 - a00) * fy
            b1 = a10 + (a11 - a10) * fy
            plsc.store_scatter(rgb_st, [r3 + c], _sigmoid(b0 + (b1 - b0) * fx))

    pltpu.sync_copy(hits_st, hits_hbm.at[pl.ds(base, RPT)])
    pltpu.sync_copy(rgb_st, rgb_hbm.at[pl.ds(base * 3, 3 * RPT)])


@functools.cache
def _get_sc_kernel():
    return functools.partial(
        pl.kernel,
        out_type=(jax.ShapeDtypeStruct((B,), jnp.float32),
                  jax.ShapeDtypeStruct((3 * B,), jnp.float32)),
        mesh=plsc.VectorSubcoreMesh(core_axis_name="c", subcore_axis_name="s",
                                    num_cores=NC, num_subcores=NS),
        compiler_params=pltpu.CompilerParams(needs_layout_passes=False,
                                             use_tc_tiling_on_sc=False),
        scratch_types=[
            pltpu.VMEM((6 * RPT,), jnp.float32),
            pltpu.VMEM((3 * NV_PAD,), jnp.float32),
            pltpu.VMEM((RPT,), jnp.int32),
            pltpu.VMEM((3 * RPT,), jnp.float32),
            pltpu.VMEM((RPT,), jnp.float32),
            pltpu.VMEM((3 * RPT,), jnp.float32),
        ],
    )(_sc_body)


def kernel(x, grid0, grid1, grid2, W_label, b_label, W_rgb, b_rgb):
    st1, ct1 = jnp.sin(x[:, 0]), jnp.cos(x[:, 0])
    st2, ct2 = jnp.sin(x[:, 2]), jnp.cos(x[:, 2])
    p1x, p1y, p1z = st1 * jnp.cos(x[:, 1]), st1 * jnp.sin(x[:, 1]), ct1
    p2x, p2y, p2z = st2 * jnp.cos(x[:, 3]), st2 * jnp.sin(x[:, 3]), ct2
    rdat = jnp.concatenate(
        [p1x, p1y, p1z, p2x - p1x, p2y - p1y, p2z - p1z])
    flds = _build_fields(grid0, grid1, grid2, W_label, b_label, W_rgb, b_rgb)
    hits_flat, rgb_flat = _get_sc_kernel()(rdat, flds)
    return hits_flat.reshape(B, 1), rgb_flat.reshape(B, 3)
